# baseline (speedup 1.0000x reference)
import functools
import math

import jax
import jax.numpy as jnp
from jax import lax
from jax.experimental import pallas as pl
from jax.experimental.pallas import tpu as pltpu
from jax.experimental.pallas import tpu_sc as plsc


def kernel(token_ids, embedding, pe):
    B, T = token_ids.shape
    V, D = embedding.shape
    pe_t = pe[0, :T, :]
    scale = math.sqrt(D)

    info = plsc.get_sparse_core_info()
    NC = info.num_cores
    NW = NC * info.num_subcores
    G = 40
    NTC = T // G
    BPW = B // NW
    NSLOT = NTC * BPW
    NB = 5

    tok_arr = (token_ids.astype(jnp.int32)
               .reshape(NW, BPW, NTC, G)
               .transpose(0, 2, 1, 3)
               .reshape(NW * NSLOT * G))

    mesh = plsc.VectorSubcoreMesh(core_axis_name="c", subcore_axis_name="s")

    @functools.partial(
        pl.kernel,
        mesh=mesh,
        out_type=jax.ShapeDtypeStruct((B * T, D), jnp.float32),
        scratch_types=[
            pltpu.VMEM((G, D), jnp.float32),
        ] + [pltpu.VMEM((G, D), jnp.float32) for _ in range(NB)]
          + [pltpu.VMEM((G,), jnp.int32) for _ in range(NB)]
          + [pltpu.SemaphoreType.DMA for _ in range(3 * NB)],
    )
    def emb_kernel(tok_hbm, emb_hbm, pe_hbm, out_hbm, pe_v, *rest):
        buf = rest[:NB]
        idx = rest[NB:2 * NB]
        sg = rest[2 * NB:3 * NB]
        ss = rest[3 * NB:4 * NB]
        si = rest[4 * NB:5 * NB]
        wid = lax.axis_index("s") * NC + lax.axis_index("c")
        b0 = wid * BPW

        def out_row0(s):
            return (b0 + s % BPW) * T + (s // BPW) * G

        def idx_copy(p, s):
            return pltpu.make_async_copy(
                tok_hbm.at[pl.ds((wid * NSLOT + s) * G, G)], idx[p], si[p])

        def gather_copy(p, s):
            return pltpu.make_async_copy(emb_hbm.at[idx[p]], buf[p], sg[p])

        def store_copy(p, s):
            return pltpu.make_async_copy(buf[p],
                                         out_hbm.at[pl.ds(out_row0(s), G)],
                                         ss[p])

        def compute(p):
            def row_body(r, _):
                for j in range(D // 16):
                    sl = pl.ds(j * 16, 16)
                    buf[p][r, sl] = buf[p][r, sl] * scale + pe_v[r, sl]
                return 0
            lax.fori_loop(0, G, row_body, 0)

        pltpu.sync_copy(pe_hbm.at[pl.ds(0, G)], pe_v)
        for p in range(4):
            idx_copy(p, p).start()
        for p in range(3):
            idx_copy(p, p).wait()
            gather_copy(p, p).start()

        def body(k, _):
            for u in range(NB):
                s = k * NB + u
                p = u

                @pl.when(jnp.logical_and(s % BPW == 0, s > 0))
                def _():
                    pltpu.sync_copy(
                        pe_hbm.at[pl.ds((s // BPW) * G, G)], pe_v)

                gather_copy(p, s).wait()
                compute(p)
                store_copy(p, s).start()

                q = (u + 3) % NB

                @pl.when(s >= 2)
                def _():
                    store_copy(q, s - 2).wait()

                @pl.when(s + 3 < NSLOT)
                def _():
                    idx_copy(q, s + 3).wait()
                    gather_copy(q, s + 3).start()

                r = (u + 4) % NB

                @pl.when(s + 4 < NSLOT)
                def _():
                    idx_copy(r, s + 4).start()
            return 0

        lax.fori_loop(0, NSLOT // NB, body, 0)
        store_copy((NSLOT - 2) % NB, NSLOT - 2).wait()
        store_copy((NSLOT - 1) % NB, NSLOT - 1).wait()

    out = emb_kernel(tok_arr, embedding, pe_t)
    return out.reshape(B, T, D)
